# Initial kernel scaffold; baseline (speedup 1.0000x reference)
import jax
import jax.numpy as jnp
from jax.experimental import pallas as pl


def kernel(q, Z, W0, b0, W1, b1):
    raise NotImplementedError("write your pallas kernel here")


# speedup vs baseline: 11.5273x; 11.5273x over previous
import functools

import jax
import jax.numpy as jnp
from jax.experimental import pallas as pl


def _mlp_block_kernel(q_ref, z_ref, w0_ref, b0_ref, w1_ref, o_ref, *, num_types, neurons):
    qb = q_ref[...]
    blk = qb.shape[0]
    th = num_types * neurons
    p = jnp.dot(qb, w0_ref[...], preferred_element_type=jnp.float32)
    g = jnp.tanh(p + b0_ref[...]) * w1_ref[...]
    z = z_ref[0, 0, :]
    col_type = jax.lax.broadcasted_iota(jnp.int32, (blk, th), 1) // neurons
    sel = jnp.where(col_type == z[:, None], g, 0.0)
    o_ref[0, 0, :] = jnp.sum(sel, axis=1)


def kernel(q, Z, W0, b0, W1, b1):
    n, d = q.shape
    num_types, _, neurons = W0.shape
    th = num_types * neurons
    blk = 1024
    grid = n // blk

    w0r = jnp.transpose(W0, (1, 0, 2)).reshape(d, th)
    b0r = b0.reshape(1, th)
    w1r = W1.reshape(1, th)
    z3 = Z.reshape(grid, 1, blk)

    f = pl.pallas_call(
        functools.partial(_mlp_block_kernel, num_types=num_types, neurons=neurons),
        grid=(grid,),
        in_specs=[
            pl.BlockSpec((blk, d), lambda i: (i, 0)),
            pl.BlockSpec((1, 1, blk), lambda i: (i, 0, 0)),
            pl.BlockSpec((d, th), lambda i: (0, 0)),
            pl.BlockSpec((1, th), lambda i: (0, 0)),
            pl.BlockSpec((1, th), lambda i: (0, 0)),
        ],
        out_specs=pl.BlockSpec((1, 1, blk), lambda i: (i, 0, 0)),
        out_shape=jax.ShapeDtypeStruct((grid, 1, blk), jnp.float32),
    )(q, z3, w0r, b0r, w1r)

    return f.reshape(n) + b1


# speedup vs baseline: 18.3110x; 1.5885x over previous
import functools

import jax
import jax.numpy as jnp
from jax.experimental import pallas as pl


def _mlp_block_kernel(q_ref, z_ref, w0_ref, b0_ref, ew_ref, b1_ref, o_ref, *, num_types):
    qb = q_ref[...]
    blk = qb.shape[0]
    pt = jax.lax.dot_general(w0_ref[...], qb, (((0,), (1,)), ((), ())),
                             preferred_element_type=jnp.float32)
    ht = jnp.tanh(pt + b0_ref[...])
    st = jnp.dot(ew_ref[...], ht, preferred_element_type=jnp.float32)
    z = z_ref[0, 0, :]
    t_iota = jax.lax.broadcasted_iota(jnp.int32, (num_types, blk), 0)
    sel = jnp.where(t_iota == z[None, :], st, 0.0)
    o_ref[0, 0, :] = jnp.sum(sel, axis=0) + b1_ref[0, 0]


def kernel(q, Z, W0, b0, W1, b1):
    n, d = q.shape
    num_types, _, neurons = W0.shape
    th = num_types * neurons
    blk = 1024
    grid = n // blk

    w0r = jnp.transpose(W0, (1, 0, 2)).reshape(d, th)
    b0c = b0.reshape(th, 1)
    eye = jnp.eye(num_types, dtype=jnp.float32)
    ewt = (eye[:, :, None] * W1[None, :, :]).reshape(num_types, th)
    z3 = Z.reshape(grid, 1, blk)
    b1a = jnp.full((1, 1), b1, dtype=jnp.float32)

    f = pl.pallas_call(
        functools.partial(_mlp_block_kernel, num_types=num_types),
        grid=(grid,),
        in_specs=[
            pl.BlockSpec((blk, d), lambda i: (i, 0)),
            pl.BlockSpec((1, 1, blk), lambda i: (i, 0, 0)),
            pl.BlockSpec((d, th), lambda i: (0, 0)),
            pl.BlockSpec((th, 1), lambda i: (0, 0)),
            pl.BlockSpec((num_types, th), lambda i: (0, 0)),
            pl.BlockSpec((1, 1), lambda i: (0, 0)),
        ],
        out_specs=pl.BlockSpec((1, 1, blk), lambda i: (i, 0, 0)),
        out_shape=jax.ShapeDtypeStruct((grid, 1, blk), jnp.float32),
    )(q, z3, w0r, b0c, ewt, b1a)

    return f.reshape(n)


# speedup vs baseline: 21.0093x; 1.1474x over previous
import functools

import jax
import jax.numpy as jnp
from jax.experimental import pallas as pl


def _mlp_block_kernel(q_ref, z_ref, w0_ref, b0_ref, ew_ref, b1_ref, o_ref, *, num_types):
    qb = q_ref[...]
    blk = qb.shape[0]
    pt = jax.lax.dot_general(w0_ref[...], qb, (((0,), (1,)), ((), ())),
                             preferred_element_type=jnp.float32)
    ht = jnp.tanh(pt + b0_ref[...])
    st = jnp.dot(ew_ref[...], ht, preferred_element_type=jnp.float32)
    z = z_ref[0, :]
    t_iota = jax.lax.broadcasted_iota(jnp.int32, (num_types, blk), 0)
    sel = jnp.where(t_iota == z[None, :], st, 0.0)
    o_ref[0, :] = jnp.sum(sel, axis=0) + b1_ref[0, 0]


def kernel(q, Z, W0, b0, W1, b1):
    n, d = q.shape
    num_types, _, neurons = W0.shape
    th = num_types * neurons
    blk = 2048
    grid = n // blk

    w0r = jnp.transpose(W0, (1, 0, 2)).reshape(d, th)
    b0c = b0.reshape(th, 1)
    eye = jnp.eye(num_types, dtype=jnp.float32)
    ewt = (eye[:, :, None] * W1[None, :, :]).reshape(num_types, th)
    z2 = Z.reshape(1, n)
    b1a = jnp.full((1, 1), b1, dtype=jnp.float32)

    f = pl.pallas_call(
        functools.partial(_mlp_block_kernel, num_types=num_types),
        grid=(grid,),
        in_specs=[
            pl.BlockSpec((blk, d), lambda i: (i, 0)),
            pl.BlockSpec((1, blk), lambda i: (0, i)),
            pl.BlockSpec((d, th), lambda i: (0, 0)),
            pl.BlockSpec((th, 1), lambda i: (0, 0)),
            pl.BlockSpec((num_types, th), lambda i: (0, 0)),
            pl.BlockSpec((1, 1), lambda i: (0, 0)),
        ],
        out_specs=pl.BlockSpec((1, blk), lambda i: (0, i)),
        out_shape=jax.ShapeDtypeStruct((1, n), jnp.float32),
    )(q, z2, w0r, b0c, ewt, b1a)

    return f.reshape(n)


# speedup vs baseline: 21.5188x; 1.0243x over previous
import functools

import jax
import jax.numpy as jnp
from jax.experimental import pallas as pl


def _mlp_block_kernel(q_ref, z_ref, w0_ref, b0_ref, ew_ref, b1_ref, o_ref, *, num_types):
    qb = q_ref[...]
    blk = qb.shape[0]
    pt = jax.lax.dot_general(w0_ref[...], qb, (((0,), (1,)), ((), ())),
                             preferred_element_type=jnp.float32)
    ht = jnp.tanh(pt + b0_ref[...])
    st = jnp.dot(ew_ref[...], ht, preferred_element_type=jnp.float32)
    z = z_ref[0, :]
    t_iota = jax.lax.broadcasted_iota(jnp.int32, (num_types, blk), 0)
    sel = jnp.where(t_iota == z[None, :], st, 0.0)
    o_ref[0, :] = jnp.sum(sel, axis=0) + b1_ref[0, 0]


def kernel(q, Z, W0, b0, W1, b1):
    n, d = q.shape
    num_types, _, neurons = W0.shape
    th = num_types * neurons
    blk = 8192
    grid = n // blk

    w0r = jnp.transpose(W0, (1, 0, 2)).reshape(d, th)
    b0c = b0.reshape(th, 1)
    eye = jnp.eye(num_types, dtype=jnp.float32)
    ewt = (eye[:, :, None] * W1[None, :, :]).reshape(num_types, th)
    z2 = Z.reshape(1, n)
    b1a = jnp.full((1, 1), b1, dtype=jnp.float32)

    f = pl.pallas_call(
        functools.partial(_mlp_block_kernel, num_types=num_types),
        grid=(grid,),
        in_specs=[
            pl.BlockSpec((blk, d), lambda i: (i, 0)),
            pl.BlockSpec((1, blk), lambda i: (0, i)),
            pl.BlockSpec((d, th), lambda i: (0, 0)),
            pl.BlockSpec((th, 1), lambda i: (0, 0)),
            pl.BlockSpec((num_types, th), lambda i: (0, 0)),
            pl.BlockSpec((1, 1), lambda i: (0, 0)),
        ],
        out_specs=pl.BlockSpec((1, blk), lambda i: (0, i)),
        out_shape=jax.ShapeDtypeStruct((1, n), jnp.float32),
    )(q, z2, w0r, b0c, ewt, b1a)

    return f.reshape(n)


# speedup vs baseline: 21.8504x; 1.0154x over previous
import functools

import jax
import jax.numpy as jnp
from jax.experimental import pallas as pl


def _mlp_block_kernel(q_ref, z_ref, w0_ref, b0_ref, ew_ref, b1_ref, o_ref, *, num_types):
    qb = q_ref[...]
    blk = qb.shape[0]
    pt = jax.lax.dot_general(w0_ref[...], qb, (((0,), (1,)), ((), ())),
                             preferred_element_type=jnp.float32)
    ht = jnp.tanh(pt + b0_ref[...])
    st = jnp.dot(ew_ref[...], ht, preferred_element_type=jnp.float32)
    z = z_ref[0, :]
    t_iota = jax.lax.broadcasted_iota(jnp.int32, (num_types, blk), 0)
    sel = jnp.where(t_iota == z[None, :], st, 0.0)
    o_ref[0, :] = jnp.sum(sel, axis=0) + b1_ref[0, 0]


def kernel(q, Z, W0, b0, W1, b1):
    n, d = q.shape
    num_types, _, neurons = W0.shape
    th = num_types * neurons
    blk = 4096
    grid = n // blk

    w0r = jnp.transpose(W0, (1, 0, 2)).reshape(d, th)
    b0c = b0.reshape(th, 1)
    eye = jnp.eye(num_types, dtype=jnp.float32)
    ewt = (eye[:, :, None] * W1[None, :, :]).reshape(num_types, th)
    z2 = Z.reshape(1, n)
    b1a = jnp.full((1, 1), b1, dtype=jnp.float32)

    f = pl.pallas_call(
        functools.partial(_mlp_block_kernel, num_types=num_types),
        grid=(grid,),
        in_specs=[
            pl.BlockSpec((blk, d), lambda i: (i, 0)),
            pl.BlockSpec((1, blk), lambda i: (0, i)),
            pl.BlockSpec((d, th), lambda i: (0, 0)),
            pl.BlockSpec((th, 1), lambda i: (0, 0)),
            pl.BlockSpec((num_types, th), lambda i: (0, 0)),
            pl.BlockSpec((1, 1), lambda i: (0, 0)),
        ],
        out_specs=pl.BlockSpec((1, blk), lambda i: (0, i)),
        out_shape=jax.ShapeDtypeStruct((1, n), jnp.float32),
    )(q, z2, w0r, b0c, ewt, b1a)

    return f.reshape(n)


# speedup vs baseline: 26.8127x; 1.2271x over previous
import functools

import jax
import jax.numpy as jnp
from jax.experimental import pallas as pl


def _mlp_block_kernel(q_ref, z_ref, w0_ref, b0_ref, w1_ref, b1_ref, o_ref, *, num_types):
    qb = q_ref[...]
    blk = qb.shape[0]
    neurons = w0_ref.shape[2]
    th = num_types * neurons

    w0r = jnp.concatenate([w0_ref[t] for t in range(num_types)], axis=1)
    b0row = jnp.concatenate([b0_ref[t:t + 1, :] for t in range(num_types)], axis=1)
    b0c = jnp.transpose(b0row)
    c_iota = jax.lax.broadcasted_iota(jnp.int32, (num_types, th), 1)
    r_iota = jax.lax.broadcasted_iota(jnp.int32, (num_types, th), 0)
    w1tile = jnp.tile(w1_ref[...], (1, num_types))
    ewt = jnp.where(c_iota // neurons == r_iota, w1tile, 0.0)

    pt = jax.lax.dot_general(w0r, qb, (((0,), (1,)), ((), ())),
                             preferred_element_type=jnp.float32)
    ht = jnp.tanh(pt + b0c)
    st = jnp.dot(ewt, ht, preferred_element_type=jnp.float32)
    z = z_ref[...]
    t_iota = jax.lax.broadcasted_iota(jnp.int32, (num_types, blk), 0)
    sel = jnp.where(t_iota == z[None, :], st, 0.0)
    o_ref[...] = jnp.sum(sel, axis=0) + b1_ref[0, 0]


def kernel(q, Z, W0, b0, W1, b1):
    n, d = q.shape
    num_types, _, neurons = W0.shape
    blk = 4096
    grid = n // blk

    b1a = jnp.full((1, 1), b1, dtype=jnp.float32)

    f = pl.pallas_call(
        functools.partial(_mlp_block_kernel, num_types=num_types),
        grid=(grid,),
        in_specs=[
            pl.BlockSpec((blk, d), lambda i: (i, 0)),
            pl.BlockSpec((blk,), lambda i: (i,)),
            pl.BlockSpec((num_types, d, neurons), lambda i: (0, 0, 0)),
            pl.BlockSpec((num_types, neurons), lambda i: (0, 0)),
            pl.BlockSpec((num_types, neurons), lambda i: (0, 0)),
            pl.BlockSpec((1, 1), lambda i: (0, 0)),
        ],
        out_specs=pl.BlockSpec((blk,), lambda i: (i,)),
        out_shape=jax.ShapeDtypeStruct((n,), jnp.float32),
    )(q, Z, W0, b0, W1, b1a)

    return f
